# baseline (speedup 1.0000x reference)
import jax
import jax.numpy as jnp
from jax.experimental import pallas as pl
from jax.experimental.pallas import tpu as pltpu

_NBLK = 32
_DEPTH_INFLIGHT = 8


def _build_phase_table(rel_embeddings):
    flat = jnp.flip(rel_embeddings, axis=0).reshape(-1)
    flat = jnp.concatenate([flat, jnp.zeros((1040,), flat.dtype)])
    rows = [
        jax.lax.dynamic_slice(
            flat, (16 * (7 - r) + 128 * (7 - e),), (65536,)
        ).reshape(512, 128)
        for e in range(8)
        for r in range(8)
    ]
    return jnp.stack(rows).reshape(8, 8, 512, 128)


def _dma_kernel(q_hbm, out_hbm, q_vmem, load_sem, sems):
    load = pltpu.make_async_copy(q_hbm, q_vmem, load_sem)
    load.start()
    load.wait()

    def _block_copy(c):
        return pltpu.make_async_copy(
            q_vmem.at[:, :, pl.ds(248 - 8 * c, 256), :],
            out_hbm.at[c],
            sems.at[jax.lax.rem(c, _DEPTH_INFLIGHT)],
        )

    def body(c, _):
        _block_copy(c).start()

        @pl.when(c >= _DEPTH_INFLIGHT - 1)
        def _():
            _block_copy(c - (_DEPTH_INFLIGHT - 1)).wait()

        return 0

    jax.lax.fori_loop(0, _NBLK, body, 0)

    def tail(c, _):
        _block_copy(c).wait()
        return 0

    jax.lax.fori_loop(_NBLK - (_DEPTH_INFLIGHT - 1), _NBLK, tail, 0)


def kernel(inputs, rel_embeddings):
    del inputs
    q = _build_phase_table(rel_embeddings)
    out = pl.pallas_call(
        _dma_kernel,
        in_specs=[pl.BlockSpec(memory_space=pl.ANY)],
        out_specs=pl.BlockSpec(memory_space=pl.ANY),
        out_shape=jax.ShapeDtypeStruct((32, 8, 8, 256, 128), jnp.float32),
        scratch_shapes=[
            pltpu.VMEM((8, 8, 512, 128), jnp.float32),
            pltpu.SemaphoreType.DMA,
            pltpu.SemaphoreType.DMA((_DEPTH_INFLIGHT,)),
        ],
    )(q)
    return out.reshape(2048, 2048, 16)


# speedup vs baseline: 1.0101x; 1.0101x over previous
import jax
import jax.numpy as jnp
from jax.experimental import pallas as pl
from jax.experimental.pallas import tpu as pltpu


def _dma_kernel(f_ref, out_hbm, q4, sem):
    f = f_ref[...]
    for p in range(8):
        if p == 0:
            plane = f[0:520, :]
        else:
            plane = jnp.concatenate(
                [f[0:520, 16 * p:], f[1:521, : 16 * p]], axis=1
            )
        r = 7 - p
        for e in range(8):
            q4[e, r] = jax.lax.slice(plane, (7 - e, 0), (519 - e, 128))

    copies = []
    for c in range(32):
        a = 248 - 8 * c
        for e in range(8):
            copies.append(
                pltpu.make_async_copy(
                    q4.at[e, :, pl.ds(a, 256), :],
                    out_hbm.at[c, e],
                    sem,
                )
            )
    for cp in copies:
        cp.start()
    for cp in copies:
        cp.wait()


def kernel(inputs, rel_embeddings):
    del inputs
    flat = jnp.flip(rel_embeddings, axis=0).reshape(-1)
    f2d = jnp.concatenate([flat, jnp.zeros((1168,), flat.dtype)]).reshape(521, 128)
    out = pl.pallas_call(
        _dma_kernel,
        in_specs=[pl.BlockSpec(memory_space=pltpu.MemorySpace.VMEM)],
        out_specs=pl.BlockSpec(memory_space=pl.ANY),
        out_shape=jax.ShapeDtypeStruct((32, 8, 8, 256, 128), jnp.float32),
        scratch_shapes=[
            pltpu.VMEM((8, 8, 512, 128), jnp.float32),
            pltpu.SemaphoreType.DMA,
        ],
    )(f2d)
    return out.reshape(2048, 2048, 16)
